# baseline (speedup 1.0000x reference)
import functools

import jax
import jax.numpy as jnp
from jax import lax
from jax.experimental import pallas as pl
from jax.experimental.pallas import tpu as pltpu
from jax.experimental.pallas import tpu_sc as plsc

N_NODES = 10000
N_PAD = 10240
D = 128
E_PAD = 327680
CHUNK = 128
N_ROWS = E_PAD // CHUNK
NC, NS = 2, 16
ROWS_DEG = N_ROWS // NS
ROWS_AGG = N_ROWS // (NC * NS)
SLICE = N_PAD // NS
ZB = 64


def _mesh():
    return plsc.VectorSubcoreMesh(core_axis_name="c", subcore_axis_name="s")


def _deg_body(idx_all, deg2, idx_v, ones_v, zrow_v, deg_s, dsem):
    c = lax.axis_index("c")
    s = lax.axis_index("s")
    for i in range(CHUNK // 16):
        ones_v[pl.ds(i * 16, 16)] = jnp.full((16,), 1.0, jnp.float32)
    for i in range(SLICE // 16):
        zrow_v[pl.ds(i * 16, 16)] = jnp.zeros((16,), jnp.float32)
    pltpu.sync_copy(zrow_v, deg_s.at[pl.ds(s * SLICE, SLICE)])
    pltpu.sync_copy(idx_all.at[c, pl.ds(s * ROWS_DEG, ROWS_DEG)], idx_v)
    plsc.subcore_barrier()

    fire = 8

    def body(g, carry):
        for t in range(fire):
            pltpu.async_copy(ones_v, deg_s.at[idx_v.at[g * fire + t]], dsem,
                             add=True)
        for t in range(fire):
            pltpu.make_async_copy(ones_v, deg_s.at[pl.ds(0, CHUNK)],
                                  dsem).wait()
        return carry

    lax.fori_loop(0, ROWS_DEG // fire, body, 0)
    plsc.subcore_barrier()
    pltpu.sync_copy(deg_s.at[pl.ds(s * SLICE, SLICE)],
                    deg2.at[c, pl.ds(s * SLICE, SLICE)])


_deg_kernel = functools.partial(
    pl.kernel,
    out_type=jax.ShapeDtypeStruct((NC, N_PAD), jnp.float32),
    mesh=_mesh(),
    scratch_types=[
        pltpu.VMEM((ROWS_DEG, CHUNK), jnp.int32),
        pltpu.VMEM((CHUNK,), jnp.float32),
        pltpu.VMEM((SLICE,), jnp.float32),
        pltpu.VMEM_SHARED((N_PAD,), jnp.float32),
        pltpu.SemaphoreType.DMA,
    ],
)(_deg_body)


NBUF = 2


def _agg_body(x_hbm, idx_all, agg2, didx, agg_s,
              buf0, buf1, sib0, sib1, ga, gb, ia, ib):
    bufs, sidxb = (buf0, buf1), (sib0, sib1)
    gsems, isems = (ga, gb), (ia, ib)
    c = lax.axis_index("c")
    s = lax.axis_index("s")
    base = (c * NS + s) * ROWS_AGG
    pltpu.sync_copy(idx_all.at[1, pl.ds(base, ROWS_AGG)], didx)

    def zrow(r, carry):
        for i in range(D // 16):
            buf0[r, pl.ds(i * 16, 16)] = jnp.zeros((16,), jnp.float32)
        return carry

    lax.fori_loop(0, CHUNK, zrow, 0)
    for t in range(SLICE // CHUNK):
        pltpu.sync_copy(buf0, agg_s.at[pl.ds(s * SLICE + t * CHUNK, CHUNK)])
    plsc.subcore_barrier()

    for b in range(NBUF):
        pltpu.async_copy(idx_all.at[0, base + b], sidxb[b], isems[b])
    for b in range(NBUF):
        pltpu.make_async_copy(idx_all.at[0, 0], sidxb[b], isems[b]).wait()
        pltpu.async_copy(x_hbm.at[sidxb[b]], bufs[b], gsems[b])

    def step(k, carry):
        for b in range(NBUF):
            j = NBUF * k + b
            pltpu.make_async_copy(x_hbm.at[pl.ds(0, CHUNK)], bufs[b],
                                  gsems[b]).wait()

            @pl.when(k < ROWS_AGG // NBUF - 1)
            def _():
                pltpu.async_copy(idx_all.at[0, base + j + NBUF], sidxb[b],
                                 isems[b])

            pltpu.sync_copy(bufs[b], agg_s.at[didx.at[j]], add=True)

            @pl.when(k < ROWS_AGG // NBUF - 1)
            def _():
                pltpu.make_async_copy(idx_all.at[0, 0], sidxb[b],
                                      isems[b]).wait()
                pltpu.async_copy(x_hbm.at[sidxb[b]], bufs[b], gsems[b])
        return carry

    lax.fori_loop(0, ROWS_AGG // NBUF, step, 0)
    plsc.subcore_barrier()
    pltpu.sync_copy(agg_s.at[pl.ds(s * SLICE, SLICE)],
                    agg2.at[c, pl.ds(s * SLICE, SLICE)])


_agg_kernel = functools.partial(
    pl.kernel,
    out_type=jax.ShapeDtypeStruct((NC, N_PAD, D), jnp.float32),
    mesh=_mesh(),
    scratch_types=(
        [pltpu.VMEM((ROWS_AGG, CHUNK), jnp.int32),
         pltpu.VMEM_SHARED((N_PAD, D), jnp.float32)]
        + [pltpu.VMEM((CHUNK, D), jnp.float32)] * NBUF
        + [pltpu.VMEM((CHUNK,), jnp.int32)] * NBUF
        + [pltpu.SemaphoreType.DMA] * (2 * NBUF)
    ),
)(_agg_body)


def _norm_body(h_ref, deg_ref, x_ref):
    deg = deg_ref[0, :, 0]
    norm = lax.rsqrt(jnp.maximum(deg, 1.0))
    x_ref[...] = h_ref[...] * norm[:, None]


def _norm_x(h_pad, deg3):
    return pl.pallas_call(
        _norm_body,
        grid=(N_PAD // SLICE,),
        in_specs=[
            pl.BlockSpec((SLICE, D), lambda i: (i, 0)),
            pl.BlockSpec((1, SLICE, 1), lambda i: (0, i, 0)),
        ],
        out_specs=pl.BlockSpec((SLICE, D), lambda i: (i, 0)),
        out_shape=jax.ShapeDtypeStruct((N_PAD, D), jnp.float32),
    )(h_pad, deg3)


def _head_body(agg_ref, deg_ref, wc, bc, wl, bl, wo, bo, out_ref):
    a = agg_ref[0] + agg_ref[1]
    deg = deg_ref[0, :, 0]
    a = a * lax.rsqrt(jnp.maximum(deg, 1.0))[:, None]
    t = jnp.dot(a, wc[...], preferred_element_type=jnp.float32) + bc[...]
    t = jnp.maximum(t, 0.0)
    t = jnp.dot(t, wl[...], preferred_element_type=jnp.float32) + bl[...]
    t = jnp.maximum(t, 0.0)
    out_ref[...] = (jnp.dot(t, wo[...], preferred_element_type=jnp.float32)
                    + bo[...])


HEAD_R = 400


def _head(agg2, deg3, wc, bc, wl, bl, wo, bo):
    full = pl.BlockSpec((1, D), lambda i: (0, 0))
    wspec = pl.BlockSpec((D, D), lambda i: (0, 0))
    return pl.pallas_call(
        _head_body,
        grid=(N_NODES // HEAD_R,),
        in_specs=[
            pl.BlockSpec((NC, HEAD_R, D), lambda i: (0, i, 0)),
            pl.BlockSpec((1, HEAD_R, 1), lambda i: (1, i, 0)),
            wspec, full, wspec, full, wspec, full,
        ],
        out_specs=pl.BlockSpec((HEAD_R, D), lambda i: (i, 0)),
        out_shape=jax.ShapeDtypeStruct((N_NODES, D), jnp.float32),
    )(agg2, deg3, wc, bc, wl, bl, wo, bo)


def kernel(h, edge_index, W_conv, b_conv, W_lin, b_lin, W_last, b_last):
    n, d = h.shape
    e = edge_index.shape[1]
    pad = E_PAD - e
    pad_idx = n + (jnp.arange(pad, dtype=jnp.int32) % (N_PAD - n))
    idx_all = jnp.concatenate(
        [edge_index, jnp.broadcast_to(pad_idx, (2, pad))],
        axis=1).reshape(2, N_ROWS, CHUNK)

    deg2 = _deg_kernel(idx_all)
    deg3 = deg2.reshape(NC, N_PAD, 1)
    x_pad = _norm_x(h, deg3)
    agg2 = _agg_kernel(x_pad, idx_all)
    return _head(agg2, deg3, W_conv, b_conv.reshape(1, D),
                 W_lin, b_lin.reshape(1, D), W_last, b_last.reshape(1, D))


# speedup vs baseline: 1.0010x; 1.0010x over previous
import functools

import jax
import jax.numpy as jnp
from jax import lax
from jax.experimental import pallas as pl
from jax.experimental.pallas import tpu as pltpu
from jax.experimental.pallas import tpu_sc as plsc

N_NODES = 10000
N_PAD = 10240
D = 128
CHUNK = 128
N_ROWS = 2500
NC, NS = 2, 16
ROWS_DEG = 160
W_CH = N_ROWS // (NC * NS)
SLICE = N_PAD // NS


def _mesh():
    return plsc.VectorSubcoreMesh(core_axis_name="c", subcore_axis_name="s")


def _deg_body(idx_all, deg2, idx_v, ones_v, zrow_v, deg_s, dsem):
    c = lax.axis_index("c")
    s = lax.axis_index("s")
    for i in range(CHUNK // 16):
        ones_v[pl.ds(i * 16, 16)] = jnp.full((16,), 1.0, jnp.float32)
    for i in range(SLICE // 16):
        zrow_v[pl.ds(i * 16, 16)] = jnp.zeros((16,), jnp.float32)
    pltpu.sync_copy(zrow_v, deg_s.at[pl.ds(s * SLICE, SLICE)])

    @pl.when(s < NS - 1)
    def _():
        pltpu.sync_copy(idx_all.at[c, pl.ds(s * ROWS_DEG, ROWS_DEG)], idx_v)

    @pl.when(s == NS - 1)
    def _():
        pltpu.sync_copy(idx_all.at[c, pl.ds((NS - 1) * ROWS_DEG,
                                            N_ROWS - (NS - 1) * ROWS_DEG)],
                        idx_v.at[pl.ds(0, N_ROWS - (NS - 1) * ROWS_DEG)])

    plsc.subcore_barrier()

    fire = 4
    ngroups = jnp.where(s == NS - 1,
                        (N_ROWS - (NS - 1) * ROWS_DEG) // fire,
                        ROWS_DEG // fire)

    def body(g, carry):
        for t in range(fire):
            pltpu.async_copy(ones_v, deg_s.at[idx_v.at[g * fire + t]], dsem,
                             add=True)
        for t in range(fire):
            pltpu.make_async_copy(ones_v, deg_s.at[pl.ds(0, CHUNK)],
                                  dsem).wait()
        return carry

    lax.fori_loop(0, ngroups, body, 0)
    plsc.subcore_barrier()
    pltpu.sync_copy(deg_s.at[pl.ds(s * SLICE, SLICE)],
                    deg2.at[c, pl.ds(s * SLICE, SLICE)])


_deg_kernel = functools.partial(
    pl.kernel,
    out_type=jax.ShapeDtypeStruct((NC, N_PAD), jnp.float32),
    mesh=_mesh(),
    scratch_types=[
        pltpu.VMEM((ROWS_DEG, CHUNK), jnp.int32),
        pltpu.VMEM((CHUNK,), jnp.float32),
        pltpu.VMEM((SLICE,), jnp.float32),
        pltpu.VMEM_SHARED((N_PAD,), jnp.float32),
        pltpu.SemaphoreType.DMA,
    ],
)(_deg_body)


NBUF = 2


EXTRA = N_ROWS - NC * NS * W_CH


def _agg_body(x_hbm, idx_all, agg2, agg_s,
              buf0, buf1, sib0, sib1, dib0, dib1, ga, gb, ia, ib, da, db):
    bufs, sidxb, didxb = (buf0, buf1), (sib0, sib1), (dib0, dib1)
    gsems, isems, dsems = (ga, gb), (ia, ib), (da, db)
    c = lax.axis_index("c")
    s = lax.axis_index("s")
    w = c * NS + s
    start = w * W_CH + jnp.minimum(w, EXTRA)
    nch = W_CH + (w < EXTRA).astype(jnp.int32)

    def zrow(r, carry):
        for i in range(D // 16):
            buf0[r, pl.ds(i * 16, 16)] = jnp.zeros((16,), jnp.float32)
        return carry

    lax.fori_loop(0, CHUNK, zrow, 0)
    for t in range(SLICE // CHUNK):
        pltpu.sync_copy(buf0, agg_s.at[pl.ds(s * SLICE + t * CHUNK, CHUNK)])
    plsc.subcore_barrier()

    def wait_gather(b):
        pltpu.make_async_copy(x_hbm.at[pl.ds(0, CHUNK)], bufs[b],
                              gsems[b]).wait()

    for b in range(NBUF):
        pltpu.async_copy(idx_all.at[0, start + b], sidxb[b], isems[b])
        pltpu.async_copy(idx_all.at[1, start + b], didxb[b], dsems[b])
    for b in range(NBUF):
        pltpu.make_async_copy(idx_all.at[0, 0], sidxb[b], isems[b]).wait()
        pltpu.async_copy(x_hbm.at[sidxb[b]], bufs[b], gsems[b])

    def step(k, carry):
        for b in range(NBUF):
            j = NBUF * k + b
            wait_gather(b)

            @pl.when(j + NBUF < nch)
            def _():
                pltpu.async_copy(idx_all.at[0, start + j + NBUF], sidxb[b],
                                 isems[b])

            pltpu.make_async_copy(idx_all.at[1, 0], didxb[b],
                                  dsems[b]).wait()
            pltpu.sync_copy(bufs[b], agg_s.at[didxb[b]], add=True)

            @pl.when(j + NBUF < nch)
            def _():
                pltpu.async_copy(idx_all.at[1, start + j + NBUF], didxb[b],
                                 dsems[b])
                pltpu.make_async_copy(idx_all.at[0, 0], sidxb[b],
                                      isems[b]).wait()
                pltpu.async_copy(x_hbm.at[sidxb[b]], bufs[b], gsems[b])
        return carry

    lax.fori_loop(0, W_CH // NBUF, step, 0)

    @pl.when(nch > W_CH)
    def _():
        wait_gather(0)
        pltpu.make_async_copy(idx_all.at[1, 0], didxb[0], dsems[0]).wait()
        pltpu.sync_copy(bufs[0], agg_s.at[didxb[0]], add=True)

    plsc.subcore_barrier()
    pltpu.sync_copy(agg_s.at[pl.ds(s * SLICE, SLICE)],
                    agg2.at[c, pl.ds(s * SLICE, SLICE)])


_agg_kernel = functools.partial(
    pl.kernel,
    out_type=jax.ShapeDtypeStruct((NC, N_PAD, D), jnp.float32),
    mesh=_mesh(),
    scratch_types=(
        [pltpu.VMEM_SHARED((N_PAD, D), jnp.float32)]
        + [pltpu.VMEM((CHUNK, D), jnp.float32)] * NBUF
        + [pltpu.VMEM((CHUNK,), jnp.int32)] * (2 * NBUF)
        + [pltpu.SemaphoreType.DMA] * (3 * NBUF)
    ),
)(_agg_body)


def _norm_body(h_ref, deg_ref, x_ref):
    deg = deg_ref[0, :, 0]
    norm = lax.rsqrt(jnp.maximum(deg, 1.0))
    x_ref[...] = h_ref[...] * norm[:, None]


def _norm_x(h, deg3):
    return pl.pallas_call(
        _norm_body,
        grid=(N_NODES // HEAD_R,),
        in_specs=[
            pl.BlockSpec((HEAD_R, D), lambda i: (i, 0)),
            pl.BlockSpec((1, HEAD_R, 1), lambda i: (0, i, 0)),
        ],
        out_specs=pl.BlockSpec((HEAD_R, D), lambda i: (i, 0)),
        out_shape=jax.ShapeDtypeStruct((N_NODES, D), jnp.float32),
    )(h, deg3)


def _head_body(agg_ref, deg_ref, wc, bc, wl, bl, wo, bo, out_ref):
    a = agg_ref[0] + agg_ref[1]
    deg = deg_ref[0, :, 0]
    a = a * lax.rsqrt(jnp.maximum(deg, 1.0))[:, None]
    t = jnp.dot(a, wc[...], preferred_element_type=jnp.float32) + bc[...]
    t = jnp.maximum(t, 0.0)
    t = jnp.dot(t, wl[...], preferred_element_type=jnp.float32) + bl[...]
    t = jnp.maximum(t, 0.0)
    out_ref[...] = (jnp.dot(t, wo[...], preferred_element_type=jnp.float32)
                    + bo[...])


HEAD_R = 400


def _head(agg2, deg3, wc, bc, wl, bl, wo, bo):
    full = pl.BlockSpec((1, D), lambda i: (0, 0))
    wspec = pl.BlockSpec((D, D), lambda i: (0, 0))
    return pl.pallas_call(
        _head_body,
        grid=(N_NODES // HEAD_R,),
        in_specs=[
            pl.BlockSpec((NC, HEAD_R, D), lambda i: (0, i, 0)),
            pl.BlockSpec((1, HEAD_R, 1), lambda i: (1, i, 0)),
            wspec, full, wspec, full, wspec, full,
        ],
        out_specs=pl.BlockSpec((HEAD_R, D), lambda i: (i, 0)),
        out_shape=jax.ShapeDtypeStruct((N_NODES, D), jnp.float32),
    )(agg2, deg3, wc, bc, wl, bl, wo, bo)


def kernel(h, edge_index, W_conv, b_conv, W_lin, b_lin, W_last, b_last):
    n, d = h.shape
    idx_all = edge_index.reshape(2, N_ROWS, CHUNK)

    deg2 = _deg_kernel(idx_all)
    deg3 = deg2.reshape(NC, N_PAD, 1)
    x_pad = _norm_x(h, deg3)
    agg2 = _agg_kernel(x_pad, idx_all)
    return _head(agg2, deg3, W_conv, b_conv.reshape(1, D),
                 W_lin, b_lin.reshape(1, D), W_last, b_last.reshape(1, D))


# speedup vs baseline: 1.1434x; 1.1422x over previous
import functools

import jax
import jax.numpy as jnp
from jax import lax
from jax.experimental import pallas as pl
from jax.experimental.pallas import tpu as pltpu
from jax.experimental.pallas import tpu_sc as plsc

N_NODES = 10000
N_PAD = 10240
D = 128
CHUNK = 128
N_ROWS = 2500
NC, NS = 2, 16
ROWS_DEG = 160
W_CH = N_ROWS // (NC * NS)
SLICE = N_PAD // NS


def _mesh():
    return plsc.VectorSubcoreMesh(core_axis_name="c", subcore_axis_name="s")


def _deg_body(idx_all, deg2, idx_v, ones_v, zrow_v, deg_s, dsem):
    c = lax.axis_index("c")
    s = lax.axis_index("s")
    for i in range(CHUNK // 16):
        ones_v[pl.ds(i * 16, 16)] = jnp.full((16,), 1.0, jnp.float32)
    for i in range(SLICE // 16):
        zrow_v[pl.ds(i * 16, 16)] = jnp.zeros((16,), jnp.float32)
    pltpu.sync_copy(zrow_v, deg_s.at[pl.ds(s * SLICE, SLICE)])

    @pl.when(s < NS - 1)
    def _():
        pltpu.sync_copy(idx_all.at[c, pl.ds(s * ROWS_DEG, ROWS_DEG)], idx_v)

    @pl.when(s == NS - 1)
    def _():
        pltpu.sync_copy(idx_all.at[c, pl.ds((NS - 1) * ROWS_DEG,
                                            N_ROWS - (NS - 1) * ROWS_DEG)],
                        idx_v.at[pl.ds(0, N_ROWS - (NS - 1) * ROWS_DEG)])

    plsc.subcore_barrier()

    fire = 4
    ngroups = jnp.where(s == NS - 1,
                        (N_ROWS - (NS - 1) * ROWS_DEG) // fire,
                        ROWS_DEG // fire)

    def body(g, carry):
        for t in range(fire):
            pltpu.async_copy(ones_v, deg_s.at[idx_v.at[g * fire + t]], dsem,
                             add=True)
        for t in range(fire):
            pltpu.make_async_copy(ones_v, deg_s.at[pl.ds(0, CHUNK)],
                                  dsem).wait()
        return carry

    lax.fori_loop(0, ngroups, body, 0)
    plsc.subcore_barrier()
    pltpu.sync_copy(deg_s.at[pl.ds(s * SLICE, SLICE)],
                    deg2.at[c, pl.ds(s * SLICE, SLICE)])


_deg_kernel = functools.partial(
    pl.kernel,
    out_type=jax.ShapeDtypeStruct((NC, N_PAD), jnp.float32),
    mesh=_mesh(),
    scratch_types=[
        pltpu.VMEM((ROWS_DEG, CHUNK), jnp.int32),
        pltpu.VMEM((CHUNK,), jnp.float32),
        pltpu.VMEM((SLICE,), jnp.float32),
        pltpu.VMEM_SHARED((N_PAD,), jnp.float32),
        pltpu.SemaphoreType.DMA,
    ],
)(_deg_body)


NBUF = 2


EXTRA = N_ROWS - NC * NS * W_CH


def _agg_body(x_hbm, idx_all, agg2, agg_s,
              buf0, buf1, sib0, sib1, dib0, dib1, ga, gb, ia, ib, da, db):
    bufs, sidxb, didxb = (buf0, buf1), (sib0, sib1), (dib0, dib1)
    gsems, isems, dsems = (ga, gb), (ia, ib), (da, db)
    c = lax.axis_index("c")
    s = lax.axis_index("s")
    w = c * NS + s
    start = w * W_CH + jnp.minimum(w, EXTRA)
    nch = W_CH + (w < EXTRA).astype(jnp.int32)

    def zrow(r, carry):
        for i in range(D // 16):
            buf0[r, pl.ds(i * 16, 16)] = jnp.zeros((16,), jnp.float32)
        return carry

    lax.fori_loop(0, CHUNK, zrow, 0)
    for t in range(SLICE // CHUNK):
        pltpu.async_copy(buf0, agg_s.at[pl.ds(s * SLICE + t * CHUNK, CHUNK)],
                         ga)
    for t in range(SLICE // CHUNK):
        pltpu.make_async_copy(buf0, agg_s.at[pl.ds(0, CHUNK)], ga).wait()
    plsc.subcore_barrier()

    def wait_gather(b):
        pltpu.make_async_copy(x_hbm.at[pl.ds(0, CHUNK)], bufs[b],
                              gsems[b]).wait()

    for b in range(NBUF):
        pltpu.async_copy(idx_all.at[0, start + b], sidxb[b], isems[b])
        pltpu.async_copy(idx_all.at[1, start + b], didxb[b], dsems[b])
    for b in range(NBUF):
        pltpu.make_async_copy(idx_all.at[0, 0], sidxb[b], isems[b]).wait()
        pltpu.async_copy(x_hbm.at[sidxb[b]], bufs[b], gsems[b])

    def step(k, carry):
        for b in range(NBUF):
            j = NBUF * k + b
            wait_gather(b)

            @pl.when(j + NBUF < nch)
            def _():
                pltpu.async_copy(idx_all.at[0, start + j + NBUF], sidxb[b],
                                 isems[b])

            pltpu.make_async_copy(idx_all.at[1, 0], didxb[b],
                                  dsems[b]).wait()
            pltpu.sync_copy(bufs[b], agg_s.at[didxb[b]], add=True)

            @pl.when(j + NBUF < nch)
            def _():
                pltpu.async_copy(idx_all.at[1, start + j + NBUF], didxb[b],
                                 dsems[b])
                pltpu.make_async_copy(idx_all.at[0, 0], sidxb[b],
                                      isems[b]).wait()
                pltpu.async_copy(x_hbm.at[sidxb[b]], bufs[b], gsems[b])
        return carry

    lax.fori_loop(0, W_CH // NBUF, step, 0)

    @pl.when(nch > W_CH)
    def _():
        wait_gather(0)
        pltpu.make_async_copy(idx_all.at[1, 0], didxb[0], dsems[0]).wait()
        pltpu.sync_copy(bufs[0], agg_s.at[didxb[0]], add=True)

    plsc.subcore_barrier()
    pltpu.sync_copy(agg_s.at[pl.ds(s * SLICE, SLICE)],
                    agg2.at[c, pl.ds(s * SLICE, SLICE)])


_agg_kernel = functools.partial(
    pl.kernel,
    out_type=jax.ShapeDtypeStruct((NC, N_PAD, D), jnp.float32),
    mesh=_mesh(),
    scratch_types=(
        [pltpu.VMEM_SHARED((N_PAD, D), jnp.float32)]
        + [pltpu.VMEM((CHUNK, D), jnp.float32)] * NBUF
        + [pltpu.VMEM((CHUNK,), jnp.int32)] * (2 * NBUF)
        + [pltpu.SemaphoreType.DMA] * (3 * NBUF)
    ),
)(_agg_body)


def _norm_body(h_ref, deg_ref, x_ref):
    deg = deg_ref[0, :, 0]
    norm = lax.rsqrt(jnp.maximum(deg, 1.0))
    x_ref[...] = h_ref[...] * norm[:, None]


def _norm_x(h, deg3):
    return pl.pallas_call(
        _norm_body,
        grid=(N_NODES // HEAD_R,),
        in_specs=[
            pl.BlockSpec((HEAD_R, D), lambda i: (i, 0)),
            pl.BlockSpec((1, HEAD_R, 1), lambda i: (0, i, 0)),
        ],
        out_specs=pl.BlockSpec((HEAD_R, D), lambda i: (i, 0)),
        out_shape=jax.ShapeDtypeStruct((N_NODES, D), jnp.float32),
    )(h, deg3)


def _head_body(agg_ref, deg_ref, wc, bc, wl, bl, wo, bo, out_ref):
    a = agg_ref[0] + agg_ref[1]
    deg = deg_ref[0, :, 0]
    a = a * lax.rsqrt(jnp.maximum(deg, 1.0))[:, None]
    t = jnp.dot(a, wc[...], preferred_element_type=jnp.float32) + bc[...]
    t = jnp.maximum(t, 0.0)
    t = jnp.dot(t, wl[...], preferred_element_type=jnp.float32) + bl[...]
    t = jnp.maximum(t, 0.0)
    out_ref[...] = (jnp.dot(t, wo[...], preferred_element_type=jnp.float32)
                    + bo[...])


HEAD_R = 2000


def _head(agg2, deg3, wc, bc, wl, bl, wo, bo):
    full = pl.BlockSpec((1, D), lambda i: (0, 0))
    wspec = pl.BlockSpec((D, D), lambda i: (0, 0))
    return pl.pallas_call(
        _head_body,
        grid=(N_NODES // HEAD_R,),
        in_specs=[
            pl.BlockSpec((NC, HEAD_R, D), lambda i: (0, i, 0)),
            pl.BlockSpec((1, HEAD_R, 1), lambda i: (1, i, 0)),
            wspec, full, wspec, full, wspec, full,
        ],
        out_specs=pl.BlockSpec((HEAD_R, D), lambda i: (i, 0)),
        out_shape=jax.ShapeDtypeStruct((N_NODES, D), jnp.float32),
    )(agg2, deg3, wc, bc, wl, bl, wo, bo)


def kernel(h, edge_index, W_conv, b_conv, W_lin, b_lin, W_last, b_last):
    n, d = h.shape
    idx_all = edge_index.reshape(2, N_ROWS, CHUNK)

    deg2 = _deg_kernel(idx_all)
    deg3 = deg2.reshape(NC, N_PAD, 1)
    x_pad = _norm_x(h, deg3)
    agg2 = _agg_kernel(x_pad, idx_all)
    return _head(agg2, deg3, W_conv, b_conv.reshape(1, D),
                 W_lin, b_lin.reshape(1, D), W_last, b_last.reshape(1, D))


# speedup vs baseline: 1.1507x; 1.0064x over previous
import functools

import jax
import jax.numpy as jnp
from jax import lax
from jax.experimental import pallas as pl
from jax.experimental.pallas import tpu as pltpu
from jax.experimental.pallas import tpu_sc as plsc

N_NODES = 10000
N_PAD = 10240
D = 128
CHUNK = 128
N_ROWS = 2500
NC, NS = 2, 16
ROWS_DEG = 160
W_CH = N_ROWS // (NC * NS)
SLICE = N_PAD // NS


def _mesh():
    return plsc.VectorSubcoreMesh(core_axis_name="c", subcore_axis_name="s")


def _deg_body(idx_all, deg2, idx_v, ones_v, zrow_v, deg_s, dsem):
    c = lax.axis_index("c")
    s = lax.axis_index("s")
    for i in range(CHUNK // 16):
        ones_v[pl.ds(i * 16, 16)] = jnp.full((16,), 1.0, jnp.float32)
    for i in range(SLICE // 16):
        zrow_v[pl.ds(i * 16, 16)] = jnp.zeros((16,), jnp.float32)
    pltpu.sync_copy(zrow_v, deg_s.at[pl.ds(s * SLICE, SLICE)])

    @pl.when(s < NS - 1)
    def _():
        pltpu.sync_copy(idx_all.at[c, pl.ds(s * ROWS_DEG, ROWS_DEG)], idx_v)

    @pl.when(s == NS - 1)
    def _():
        pltpu.sync_copy(idx_all.at[c, pl.ds((NS - 1) * ROWS_DEG,
                                            N_ROWS - (NS - 1) * ROWS_DEG)],
                        idx_v.at[pl.ds(0, N_ROWS - (NS - 1) * ROWS_DEG)])

    plsc.subcore_barrier()

    fire = 10
    ngroups = jnp.where(s == NS - 1,
                        (N_ROWS - (NS - 1) * ROWS_DEG) // fire,
                        ROWS_DEG // fire)

    def body(g, carry):
        for t in range(fire):
            pltpu.async_copy(ones_v, deg_s.at[idx_v.at[g * fire + t]], dsem,
                             add=True)
        for t in range(fire):
            pltpu.make_async_copy(ones_v, deg_s.at[pl.ds(0, CHUNK)],
                                  dsem).wait()
        return carry

    lax.fori_loop(0, ngroups, body, 0)
    plsc.subcore_barrier()
    pltpu.sync_copy(deg_s.at[pl.ds(s * SLICE, SLICE)],
                    deg2.at[c, pl.ds(s * SLICE, SLICE)])


_deg_kernel = functools.partial(
    pl.kernel,
    out_type=jax.ShapeDtypeStruct((NC, N_PAD), jnp.float32),
    mesh=_mesh(),
    scratch_types=[
        pltpu.VMEM((ROWS_DEG, CHUNK), jnp.int32),
        pltpu.VMEM((CHUNK,), jnp.float32),
        pltpu.VMEM((SLICE,), jnp.float32),
        pltpu.VMEM_SHARED((N_PAD,), jnp.float32),
        pltpu.SemaphoreType.DMA,
    ],
)(_deg_body)


NBUF = 2


EXTRA = N_ROWS - NC * NS * W_CH


def _agg_body(x_hbm, idx_all, agg2, agg_s,
              buf0, buf1, sib0, sib1, dib0, dib1, ga, gb, ia, ib, da, db):
    bufs, sidxb, didxb = (buf0, buf1), (sib0, sib1), (dib0, dib1)
    gsems, isems, dsems = (ga, gb), (ia, ib), (da, db)
    c = lax.axis_index("c")
    s = lax.axis_index("s")
    w = c * NS + s
    start = w * W_CH + jnp.minimum(w, EXTRA)
    nch = W_CH + (w < EXTRA).astype(jnp.int32)

    def zrow(r, carry):
        for i in range(D // 16):
            buf0[r, pl.ds(i * 16, 16)] = jnp.zeros((16,), jnp.float32)
        return carry

    lax.fori_loop(0, CHUNK, zrow, 0)
    for t in range(SLICE // CHUNK):
        pltpu.async_copy(buf0, agg_s.at[pl.ds(s * SLICE + t * CHUNK, CHUNK)],
                         ga)
    for t in range(SLICE // CHUNK):
        pltpu.make_async_copy(buf0, agg_s.at[pl.ds(0, CHUNK)], ga).wait()
    plsc.subcore_barrier()

    def wait_gather(b):
        pltpu.make_async_copy(x_hbm.at[pl.ds(0, CHUNK)], bufs[b],
                              gsems[b]).wait()

    for b in range(NBUF):
        pltpu.async_copy(idx_all.at[0, start + b], sidxb[b], isems[b])
        pltpu.async_copy(idx_all.at[1, start + b], didxb[b], dsems[b])
    for b in range(NBUF):
        pltpu.make_async_copy(idx_all.at[0, 0], sidxb[b], isems[b]).wait()
        pltpu.async_copy(x_hbm.at[sidxb[b]], bufs[b], gsems[b])

    def step(k, carry):
        for b in range(NBUF):
            j = NBUF * k + b
            wait_gather(b)

            @pl.when(j + NBUF < nch)
            def _():
                pltpu.async_copy(idx_all.at[0, start + j + NBUF], sidxb[b],
                                 isems[b])

            pltpu.make_async_copy(idx_all.at[1, 0], didxb[b],
                                  dsems[b]).wait()
            pltpu.sync_copy(bufs[b], agg_s.at[didxb[b]], add=True)

            @pl.when(j + NBUF < nch)
            def _():
                pltpu.async_copy(idx_all.at[1, start + j + NBUF], didxb[b],
                                 dsems[b])
                pltpu.make_async_copy(idx_all.at[0, 0], sidxb[b],
                                      isems[b]).wait()
                pltpu.async_copy(x_hbm.at[sidxb[b]], bufs[b], gsems[b])
        return carry

    lax.fori_loop(0, W_CH // NBUF, step, 0)

    @pl.when(nch > W_CH)
    def _():
        wait_gather(0)
        pltpu.make_async_copy(idx_all.at[1, 0], didxb[0], dsems[0]).wait()
        pltpu.sync_copy(bufs[0], agg_s.at[didxb[0]], add=True)

    plsc.subcore_barrier()
    pltpu.sync_copy(agg_s.at[pl.ds(s * SLICE, SLICE)],
                    agg2.at[c, pl.ds(s * SLICE, SLICE)])


_agg_kernel = functools.partial(
    pl.kernel,
    out_type=jax.ShapeDtypeStruct((NC, N_PAD, D), jnp.float32),
    mesh=_mesh(),
    scratch_types=(
        [pltpu.VMEM_SHARED((N_PAD, D), jnp.float32)]
        + [pltpu.VMEM((CHUNK, D), jnp.float32)] * NBUF
        + [pltpu.VMEM((CHUNK,), jnp.int32)] * (2 * NBUF)
        + [pltpu.SemaphoreType.DMA] * (3 * NBUF)
    ),
)(_agg_body)


def _norm_body(h_ref, deg_ref, x_ref):
    deg = deg_ref[0, :, 0]
    norm = lax.rsqrt(jnp.maximum(deg, 1.0))
    x_ref[...] = h_ref[...] * norm[:, None]


def _norm_x(h, deg3):
    return pl.pallas_call(
        _norm_body,
        grid=(N_NODES // HEAD_R,),
        in_specs=[
            pl.BlockSpec((HEAD_R, D), lambda i: (i, 0)),
            pl.BlockSpec((1, HEAD_R, 1), lambda i: (0, i, 0)),
        ],
        out_specs=pl.BlockSpec((HEAD_R, D), lambda i: (i, 0)),
        out_shape=jax.ShapeDtypeStruct((N_NODES, D), jnp.float32),
    )(h, deg3)


def _head_body(agg_ref, deg_ref, wc, bc, wl, bl, wo, bo, out_ref):
    a = agg_ref[0] + agg_ref[1]
    deg = deg_ref[0, :, 0]
    a = a * lax.rsqrt(jnp.maximum(deg, 1.0))[:, None]
    t = jnp.dot(a, wc[...], preferred_element_type=jnp.float32) + bc[...]
    t = jnp.maximum(t, 0.0)
    t = jnp.dot(t, wl[...], preferred_element_type=jnp.float32) + bl[...]
    t = jnp.maximum(t, 0.0)
    out_ref[...] = (jnp.dot(t, wo[...], preferred_element_type=jnp.float32)
                    + bo[...])


HEAD_R = 2000


def _head(agg2, deg3, wc, bc, wl, bl, wo, bo):
    full = pl.BlockSpec((1, D), lambda i: (0, 0))
    wspec = pl.BlockSpec((D, D), lambda i: (0, 0))
    return pl.pallas_call(
        _head_body,
        grid=(N_NODES // HEAD_R,),
        in_specs=[
            pl.BlockSpec((NC, HEAD_R, D), lambda i: (0, i, 0)),
            pl.BlockSpec((1, HEAD_R, 1), lambda i: (1, i, 0)),
            wspec, full, wspec, full, wspec, full,
        ],
        out_specs=pl.BlockSpec((HEAD_R, D), lambda i: (i, 0)),
        out_shape=jax.ShapeDtypeStruct((N_NODES, D), jnp.float32),
    )(agg2, deg3, wc, bc, wl, bl, wo, bo)


def kernel(h, edge_index, W_conv, b_conv, W_lin, b_lin, W_last, b_last):
    n, d = h.shape
    idx_all = edge_index.reshape(2, N_ROWS, CHUNK)

    deg2 = _deg_kernel(idx_all)
    deg3 = deg2.reshape(NC, N_PAD, 1)
    x_pad = _norm_x(h, deg3)
    agg2 = _agg_kernel(x_pad, idx_all)
    return _head(agg2, deg3, W_conv, b_conv.reshape(1, D),
                 W_lin, b_lin.reshape(1, D), W_last, b_last.reshape(1, D))
